# Initial kernel scaffold; baseline (speedup 1.0000x reference)
import jax
import jax.numpy as jnp
from jax.experimental import pallas as pl


def kernel(inputs, params, edge_index):
    raise NotImplementedError("write your pallas kernel here")


# speedup vs baseline: 2.5959x; 2.5959x over previous
import functools

import jax
import jax.numpy as jnp
from jax import lax
from jax.experimental import pallas as pl
from jax.experimental.pallas import tpu as pltpu
from jax.experimental.pallas import tpu_sc as plsc

_N = 10000
_E = 320000
_C = 128
_M = 64
_F32 = jnp.float32

_NW = 32
_PT = _E // _NW
_GW = 80
_SK = 80
_SG = 5
_PACK = 4

_DOT = functools.partial(jnp.dot, preferred_element_type=_F32,
                         precision=lax.Precision.HIGHEST)


def _DOTBF(x, w):
    return jnp.dot(x, w, preferred_element_type=_F32,
                   precision=lax.Precision.HIGHEST)

_SC_PARAMS = pltpu.CompilerParams(use_tc_tiling_on_sc=False)


def _relu(x):
    return jnp.maximum(x, 0.0)


def _full_spec(shape):
    nd = len(shape)
    return pl.BlockSpec(shape, lambda i, _nd=nd: (0,) * _nd)


def _mlp_refs(x, wrefs, brefs, relu_last=False):
    n = len(wrefs)
    for i, (w, b) in enumerate(zip(wrefs, brefs)):
        x = _DOT(x, w[:]) + b[:]
        if i < n - 1 or relu_last:
            x = _relu(x)
    return x


def _encode(x, enc_ws, enc_bs, w1a, w1b, b1):
    nl = len(enc_ws)
    rows = 1000
    grid = _N // rows

    def body(x_ref, *refs):
        ws = refs[:nl]
        bs = refs[nl:2 * nl]
        w1a_r, w1b_r, b1_r = refs[2 * nl:2 * nl + 3]
        h_ref, p_ref, q_ref = refs[2 * nl + 3:]
        h = _mlp_refs(x_ref[:], ws, bs)
        h_ref[:] = h
        p_ref[:] = _DOTBF(h, w1a_r[:])
        q_ref[:] = _DOTBF(h, w1b_r[:]) + b1_r[:]

    in_specs = [pl.BlockSpec((rows, x.shape[1]), lambda i: (i, 0))]
    in_specs += [_full_spec(w.shape) for w in enc_ws]
    in_specs += [_full_spec(b.shape) for b in enc_bs]
    in_specs += [_full_spec(w1a.shape), _full_spec(w1b.shape), _full_spec(b1.shape)]
    return pl.pallas_call(
        body,
        grid=(grid,),
        in_specs=in_specs,
        out_specs=[pl.BlockSpec((rows, _C), lambda i: (i, 0)),
                   pl.BlockSpec((rows, _M), lambda i: (i, 0)),
                   pl.BlockSpec((rows, _M), lambda i: (i, 0))],
        out_shape=[jax.ShapeDtypeStruct((_N, _C), _F32),
                   jax.ShapeDtypeStruct((_N, _M), _F32),
                   jax.ShapeDtypeStruct((_N, _M), _F32)],
    )(x, *enc_ws, *enc_bs, w1a, w1b, b1)


def _gather2(p, q, src2, dst2):
    grid = _E // _GW
    mesh = plsc.VectorSubcoreMesh(core_axis_name="c", subcore_axis_name="s")

    @functools.partial(
        pl.kernel,
        out_type=(jax.ShapeDtypeStruct((_E, _M), _F32),
                  jax.ShapeDtypeStruct((_E, _M), _F32)),
        mesh=mesh,
        compiler_params=_SC_PARAMS)
    def kern(p_hbm, q_hbm, s_hbm, d_hbm, oa_hbm, ob_hbm):
        def body(s_v, d_v, oa_v, ob_v):
            pltpu.sync_copy(p_hbm.at[s_v.at[0]], oa_v)
            pltpu.sync_copy(q_hbm.at[d_v.at[0]], ob_v)

        pltpu.emit_pipeline(
            body,
            grid=(grid,),
            in_specs=[pl.BlockSpec((1, _GW), lambda i: (i, 0)),
                      pl.BlockSpec((1, _GW), lambda i: (i, 0))],
            out_specs=[pl.BlockSpec((_GW, _M), lambda i: (i, 0)),
                       pl.BlockSpec((_GW, _M), lambda i: (i, 0))],
            core_axis_name=("c", "s"),
            dimension_semantics=(pltpu.PARALLEL,),
        )(s_hbm, d_hbm, oa_hbm, ob_hbm)

    return kern(p, q, src2, dst2)


def _edge_mlp(xa4, xb4, w2d, b2d, w3d, b3d, w4d, b4d):
    rows = 2000
    n4 = _E // _PACK
    grid = n4 // rows
    wide = _M * _PACK

    def body(xa_ref, xb_ref, w2, b2, w3, b3, w4, b4, o_ref):
        x = _relu(xa_ref[:] + xb_ref[:])
        x = _relu(_DOTBF(x, w2[:]) + b2[:])
        x = _relu(_DOTBF(x, w3[:]) + b3[:])
        o_ref[:] = _DOTBF(x, w4[:]) + b4[:]

    data_spec = pl.BlockSpec((rows, wide), lambda i: (i, 0))
    return pl.pallas_call(
        body,
        grid=(grid,),
        in_specs=[data_spec, data_spec,
                  _full_spec(w2d.shape), _full_spec(b2d.shape),
                  _full_spec(w3d.shape), _full_spec(b3d.shape),
                  _full_spec(w4d.shape), _full_spec(b4d.shape)],
        out_specs=data_spec,
        out_shape=jax.ShapeDtypeStruct((n4, wide), _F32),
    )(xa4, xb4, w2d, b2d, w3d, b3d, w4d, b4d)


def _segment_sum2(m, dst3, zrows):
    nchunk = _PT // _SK
    ngroup = nchunk // _SG
    nrow = _N // 16
    mesh = plsc.VectorSubcoreMesh(core_axis_name="c", subcore_axis_name="s")

    @functools.partial(
        pl.kernel,
        out_type=jax.ShapeDtypeStruct((2, _N, _M), _F32),
        mesh=mesh,
        compiler_params=_SC_PARAMS,
        scratch_types=[pltpu.VMEM((_SG * _SK, _M), _F32),
                       pltpu.VMEM((_SG, _SK), jnp.int32),
                       pltpu.VMEM_SHARED((_N, _M), _F32)])
    def kern(m_hbm, d_hbm, z_hbm, o_hbm, mbuf, ibuf, acc):
        cid = lax.axis_index("c")
        sid = lax.axis_index("s")
        wid = cid * 16 + sid
        pltpu.sync_copy(z_hbm, acc.at[pl.ds(sid * nrow, nrow)])
        plsc.subcore_barrier()

        @pl.loop(0, ngroup)
        def _(g):
            row0 = wid * _PT + g * (_SG * _SK)
            pltpu.sync_copy(m_hbm.at[pl.ds(row0, _SG * _SK)], mbuf)
            pltpu.sync_copy(d_hbm.at[wid, pl.ds(g * _SG, _SG)], ibuf)
            for j in range(_SG):
                pltpu.sync_copy(mbuf.at[pl.ds(j * _SK, _SK)],
                                acc.at[ibuf.at[j]], add=True)

        plsc.subcore_barrier()
        pltpu.sync_copy(acc.at[pl.ds(sid * nrow, nrow)],
                        o_hbm.at[cid, pl.ds(sid * nrow, nrow)])

    return kern(m, dst3, zrows)


def _update(h, agg2, upd_ws, upd_bs, v1a, v1b, b1u, nxt):
    rows = 1000
    grid = _N // rows
    tail_ws = upd_ws[1:]
    tail_bs = upd_bs[1:]
    nt = len(tail_ws)

    def body(h_ref, a_ref, v1a_r, v1b_r, b1_r, *refs):
        ws = refs[:nt]
        bs = refs[nt:2 * nt]
        rest = refs[2 * nt:]
        hx = h_ref[:]
        agg = a_ref[0] + a_ref[1]
        t = _relu(_DOT(hx, v1a_r[:]) + _DOT(agg, v1b_r[:]) + b1_r[:])
        for i, (w, b) in enumerate(zip(ws, bs)):
            t = _DOT(t, w[:]) + b[:]
            if i < nt - 1:
                t = _relu(t)
        hn = hx + t
        if nxt is not None:
            nw1a, nw1b, nb1, h_out, p_out, q_out = rest
            h_out[:] = hn
            p_out[:] = _DOTBF(hn, nw1a[:])
            q_out[:] = _DOTBF(hn, nw1b[:]) + nb1[:]
        else:
            (h_out,) = rest
            h_out[:] = hn

    in_specs = [pl.BlockSpec((rows, _C), lambda i: (i, 0)),
                pl.BlockSpec((2, rows, _M), lambda i: (0, i, 0)),
                _full_spec(v1a.shape), _full_spec(v1b.shape), _full_spec(b1u.shape)]
    in_specs += [_full_spec(w.shape) for w in tail_ws]
    in_specs += [_full_spec(b.shape) for b in tail_bs]
    args = [h, agg2, v1a, v1b, b1u, *tail_ws, *tail_bs]
    out_specs = [pl.BlockSpec((rows, _C), lambda i: (i, 0))]
    out_shape = [jax.ShapeDtypeStruct((_N, _C), _F32)]
    if nxt is not None:
        nw1a, nw1b, nb1 = nxt
        in_specs += [_full_spec(nw1a.shape), _full_spec(nw1b.shape),
                     _full_spec(nb1.shape)]
        args += [nw1a, nw1b, nb1]
        out_specs += [pl.BlockSpec((rows, _M), lambda i: (i, 0)),
                      pl.BlockSpec((rows, _M), lambda i: (i, 0))]
        out_shape += [jax.ShapeDtypeStruct((_N, _M), _F32),
                      jax.ShapeDtypeStruct((_N, _M), _F32)]
    return pl.pallas_call(
        body,
        grid=(grid,),
        in_specs=in_specs,
        out_specs=out_specs,
        out_shape=out_shape,
    )(*args)


def _decode(h, dec_ws, dec_bs):
    nl = len(dec_ws)
    rows = 1000
    grid = _N // rows

    def body(h_ref, *refs):
        ws = refs[:nl]
        bs = refs[nl:2 * nl]
        o_ref = refs[2 * nl]
        o_ref[:] = _mlp_refs(h_ref[:], ws, bs)

    in_specs = [pl.BlockSpec((rows, _C), lambda i: (i, 0))]
    in_specs += [_full_spec(w.shape) for w in dec_ws]
    in_specs += [_full_spec(b.shape) for b in dec_bs]
    return pl.pallas_call(
        body,
        grid=(grid,),
        in_specs=in_specs,
        out_specs=pl.BlockSpec((rows, _C), lambda i: (i, 0)),
        out_shape=jax.ShapeDtypeStruct((_N, _C), _F32),
    )(h, *dec_ws, *dec_bs)

def kernel(inputs, params, edge_index):
    enc = params["enc"]
    dec = params["dec"]
    msg = params["msg"]
    upd = params["upd"]
    steps = len(msg)

    src_i = edge_index[0]
    dst_i = edge_index[1]
    src2 = src_i.reshape(_E // _GW, _GW)
    dst2 = dst_i.reshape(_E // _GW, _GW)
    dst3 = dst_i.reshape(_NW, _PT // _SK, _SK)
    zrows = jnp.zeros((_N // 16, _M), _F32)

    enc_ws = [w for w, _ in enc]
    enc_bs = [b.reshape(1, -1) for _, b in enc]
    dec_ws = [w for w, _ in dec[:-1]]
    dec_bs = [b.reshape(1, -1) for _, b in dec[:-1]]
    wlast, blast = dec[-1]
    out_ch = wlast.shape[1]
    dec_ws.append(jnp.pad(wlast, ((0, 0), (0, _C - out_ch))))
    dec_bs.append(jnp.pad(blast, (0, _C - out_ch)).reshape(1, -1))

    msplit = []
    for s in range(steps):
        w1, b1 = msg[s][0]
        msplit.append((w1[:_C], w1[_C:], b1.reshape(1, -1)))

    eye4 = jnp.eye(_PACK, dtype=_F32)
    mtail = []
    for s in range(steps):
        packed = []
        for w, b in msg[s][1:]:
            packed.append(jnp.kron(eye4, w))
            packed.append(jnp.tile(b, _PACK).reshape(1, -1))
        mtail.append(packed)

    h, p, q = _encode(inputs, enc_ws, enc_bs, *msplit[0])
    for s in range(steps):
        xa, xb = _gather2(p, q, src2, dst2)
        xa4 = xa.reshape(_E // _PACK, _M * _PACK)
        xb4 = xb.reshape(_E // _PACK, _M * _PACK)
        m4 = _edge_mlp(xa4, xb4, *mtail[s])
        agg2 = _segment_sum2(m4.reshape(_E, _M), dst3, zrows)
        upd_ws = [w for w, _ in upd[s]]
        upd_bs = [b.reshape(1, -1) for _, b in upd[s]]
        v1a = upd_ws[0][:_C]
        v1b = upd_ws[0][_C:]
        nxt = msplit[s + 1] if s + 1 < steps else None
        res = _update(h, agg2, upd_ws, upd_bs, v1a, v1b, upd_bs[0], nxt)
        if nxt is not None:
            h, p, q = res
        else:
            (h,) = res
    out = _decode(h, dec_ws, dec_bs)
    return out[:, :out_ch]


# speedup vs baseline: 3.3778x; 1.3012x over previous
import functools

import jax
import jax.numpy as jnp
from jax import lax
from jax.experimental import pallas as pl
from jax.experimental.pallas import tpu as pltpu
from jax.experimental.pallas import tpu_sc as plsc

_N = 10000
_E = 320000
_C = 128
_M = 64
_F32 = jnp.float32

_NW = 32
_PT = _E // _NW
_GW = 160
_SK = 80
_SG = 5
_PACK = 4

def _DOT(x, w):
    return jnp.dot(x.astype(jnp.bfloat16), w.astype(jnp.bfloat16),
                   preferred_element_type=_F32)


_DOTBF = _DOT

_SC_PARAMS = pltpu.CompilerParams(use_tc_tiling_on_sc=False)


def _relu(x):
    return jnp.maximum(x, 0.0)


def _full_spec(shape):
    nd = len(shape)
    return pl.BlockSpec(shape, lambda i, _nd=nd: (0,) * _nd)


def _mlp_refs(x, wrefs, brefs, relu_last=False):
    n = len(wrefs)
    for i, (w, b) in enumerate(zip(wrefs, brefs)):
        x = _DOT(x, w[:]) + b[:]
        if i < n - 1 or relu_last:
            x = _relu(x)
    return x


def _encode(x, enc_ws, enc_bs, w1a, w1b, b1):
    nl = len(enc_ws)
    rows = 1000
    grid = _N // rows

    def body(x_ref, *refs):
        ws = refs[:nl]
        bs = refs[nl:2 * nl]
        w1a_r, w1b_r, b1_r = refs[2 * nl:2 * nl + 3]
        h_ref, p_ref, q_ref = refs[2 * nl + 3:]
        h = _mlp_refs(x_ref[:], ws, bs)
        h_ref[:] = h
        p_ref[:] = _DOTBF(h, w1a_r[:])
        q_ref[:] = _DOTBF(h, w1b_r[:]) + b1_r[:]

    in_specs = [pl.BlockSpec((rows, x.shape[1]), lambda i: (i, 0))]
    in_specs += [_full_spec(w.shape) for w in enc_ws]
    in_specs += [_full_spec(b.shape) for b in enc_bs]
    in_specs += [_full_spec(w1a.shape), _full_spec(w1b.shape), _full_spec(b1.shape)]
    return pl.pallas_call(
        body,
        grid=(grid,),
        in_specs=in_specs,
        out_specs=[pl.BlockSpec((rows, _C), lambda i: (i, 0)),
                   pl.BlockSpec((rows, _M), lambda i: (i, 0)),
                   pl.BlockSpec((rows, _M), lambda i: (i, 0))],
        out_shape=[jax.ShapeDtypeStruct((_N, _C), _F32),
                   jax.ShapeDtypeStruct((_N, _M), _F32),
                   jax.ShapeDtypeStruct((_N, _M), _F32)],
    )(x, *enc_ws, *enc_bs, w1a, w1b, b1)


def _gather2(p, q, src2, dst2):
    grid = _E // _GW
    mesh = plsc.VectorSubcoreMesh(core_axis_name="c", subcore_axis_name="s")

    @functools.partial(
        pl.kernel,
        out_type=(jax.ShapeDtypeStruct((_E, _M), _F32),
                  jax.ShapeDtypeStruct((_E, _M), _F32)),
        mesh=mesh,
        compiler_params=_SC_PARAMS)
    def kern(p_hbm, q_hbm, s_hbm, d_hbm, oa_hbm, ob_hbm):
        def body(s_v, d_v, oa_v, ob_v):
            pltpu.sync_copy(p_hbm.at[s_v.at[0]], oa_v)
            pltpu.sync_copy(q_hbm.at[d_v.at[0]], ob_v)

        pltpu.emit_pipeline(
            body,
            grid=(grid,),
            in_specs=[pl.BlockSpec((1, _GW), lambda i: (i, 0)),
                      pl.BlockSpec((1, _GW), lambda i: (i, 0))],
            out_specs=[pl.BlockSpec((_GW, _M), lambda i: (i, 0)),
                       pl.BlockSpec((_GW, _M), lambda i: (i, 0))],
            core_axis_name=("c", "s"),
            dimension_semantics=(pltpu.PARALLEL,),
        )(s_hbm, d_hbm, oa_hbm, ob_hbm)

    return kern(p, q, src2, dst2)


def _edge_mlp(xa4, xb4, w2d, b2d, w3d, b3d, w4d, b4d):
    rows = 2000
    n4 = _E // _PACK
    grid = n4 // rows
    wide = _M * _PACK

    def body(xa_ref, xb_ref, w2, b2, w3, b3, w4, b4, o_ref):
        x = _relu(xa_ref[:] + xb_ref[:])
        x = _relu(_DOTBF(x, w2[:]) + b2[:])
        x = _relu(_DOTBF(x, w3[:]) + b3[:])
        o_ref[:] = _DOTBF(x, w4[:]) + b4[:]

    data_spec = pl.BlockSpec((rows, wide), lambda i: (i, 0))
    return pl.pallas_call(
        body,
        grid=(grid,),
        in_specs=[data_spec, data_spec,
                  _full_spec(w2d.shape), _full_spec(b2d.shape),
                  _full_spec(w3d.shape), _full_spec(b3d.shape),
                  _full_spec(w4d.shape), _full_spec(b4d.shape)],
        out_specs=data_spec,
        out_shape=jax.ShapeDtypeStruct((n4, wide), _F32),
    )(xa4, xb4, w2d, b2d, w3d, b3d, w4d, b4d)


def _segment_sum2(m, dst3, zrows):
    nchunk = _PT // _SK
    ngroup = nchunk // _SG
    nrow = _N // 16
    mesh = plsc.VectorSubcoreMesh(core_axis_name="c", subcore_axis_name="s")

    @functools.partial(
        pl.kernel,
        out_type=jax.ShapeDtypeStruct((2, _N, _M), _F32),
        mesh=mesh,
        compiler_params=_SC_PARAMS,
        scratch_types=[pltpu.VMEM((_SG * _SK, _M), _F32),
                       pltpu.VMEM((_SG, _SK), jnp.int32),
                       pltpu.VMEM_SHARED((_N, _M), _F32)])
    def kern(m_hbm, d_hbm, z_hbm, o_hbm, mbuf, ibuf, acc):
        cid = lax.axis_index("c")
        sid = lax.axis_index("s")
        wid = cid * 16 + sid
        pltpu.sync_copy(z_hbm, acc.at[pl.ds(sid * nrow, nrow)])
        plsc.subcore_barrier()

        @pl.loop(0, ngroup)
        def _(g):
            row0 = wid * _PT + g * (_SG * _SK)
            pltpu.sync_copy(m_hbm.at[pl.ds(row0, _SG * _SK)], mbuf)
            pltpu.sync_copy(d_hbm.at[wid, pl.ds(g * _SG, _SG)], ibuf)
            for j in range(_SG):
                pltpu.sync_copy(mbuf.at[pl.ds(j * _SK, _SK)],
                                acc.at[ibuf.at[j]], add=True)

        plsc.subcore_barrier()
        pltpu.sync_copy(acc.at[pl.ds(sid * nrow, nrow)],
                        o_hbm.at[cid, pl.ds(sid * nrow, nrow)])

    return kern(m, dst3, zrows)


def _update(h, agg2, upd_ws, upd_bs, v1a, v1b, b1u, nxt):
    rows = 1000
    grid = _N // rows
    tail_ws = upd_ws[1:]
    tail_bs = upd_bs[1:]
    nt = len(tail_ws)

    def body(h_ref, a_ref, v1a_r, v1b_r, b1_r, *refs):
        ws = refs[:nt]
        bs = refs[nt:2 * nt]
        rest = refs[2 * nt:]
        hx = h_ref[:]
        agg = a_ref[0] + a_ref[1]
        t = _relu(_DOT(hx, v1a_r[:]) + _DOT(agg, v1b_r[:]) + b1_r[:])
        for i, (w, b) in enumerate(zip(ws, bs)):
            t = _DOT(t, w[:]) + b[:]
            if i < nt - 1:
                t = _relu(t)
        hn = hx + t
        if nxt is not None:
            nw1a, nw1b, nb1, h_out, p_out, q_out = rest
            h_out[:] = hn
            p_out[:] = _DOTBF(hn, nw1a[:])
            q_out[:] = _DOTBF(hn, nw1b[:]) + nb1[:]
        else:
            (h_out,) = rest
            h_out[:] = hn

    in_specs = [pl.BlockSpec((rows, _C), lambda i: (i, 0)),
                pl.BlockSpec((2, rows, _M), lambda i: (0, i, 0)),
                _full_spec(v1a.shape), _full_spec(v1b.shape), _full_spec(b1u.shape)]
    in_specs += [_full_spec(w.shape) for w in tail_ws]
    in_specs += [_full_spec(b.shape) for b in tail_bs]
    args = [h, agg2, v1a, v1b, b1u, *tail_ws, *tail_bs]
    out_specs = [pl.BlockSpec((rows, _C), lambda i: (i, 0))]
    out_shape = [jax.ShapeDtypeStruct((_N, _C), _F32)]
    if nxt is not None:
        nw1a, nw1b, nb1 = nxt
        in_specs += [_full_spec(nw1a.shape), _full_spec(nw1b.shape),
                     _full_spec(nb1.shape)]
        args += [nw1a, nw1b, nb1]
        out_specs += [pl.BlockSpec((rows, _M), lambda i: (i, 0)),
                      pl.BlockSpec((rows, _M), lambda i: (i, 0))]
        out_shape += [jax.ShapeDtypeStruct((_N, _M), _F32),
                      jax.ShapeDtypeStruct((_N, _M), _F32)]
    return pl.pallas_call(
        body,
        grid=(grid,),
        in_specs=in_specs,
        out_specs=out_specs,
        out_shape=out_shape,
    )(*args)


def _decode(h, dec_ws, dec_bs):
    nl = len(dec_ws)
    rows = 1000
    grid = _N // rows

    def body(h_ref, *refs):
        ws = refs[:nl]
        bs = refs[nl:2 * nl]
        o_ref = refs[2 * nl]
        o_ref[:] = _mlp_refs(h_ref[:], ws, bs)

    in_specs = [pl.BlockSpec((rows, _C), lambda i: (i, 0))]
    in_specs += [_full_spec(w.shape) for w in dec_ws]
    in_specs += [_full_spec(b.shape) for b in dec_bs]
    return pl.pallas_call(
        body,
        grid=(grid,),
        in_specs=in_specs,
        out_specs=pl.BlockSpec((rows, _C), lambda i: (i, 0)),
        out_shape=jax.ShapeDtypeStruct((_N, _C), _F32),
    )(h, *dec_ws, *dec_bs)

def kernel(inputs, params, edge_index):
    enc = params["enc"]
    dec = params["dec"]
    msg = params["msg"]
    upd = params["upd"]
    steps = len(msg)

    src_i = edge_index[0]
    dst_i = edge_index[1]
    src2 = src_i.reshape(_E // _GW, _GW)
    dst2 = dst_i.reshape(_E // _GW, _GW)
    dst3 = dst_i.reshape(_NW, _PT // _SK, _SK)
    zrows = jnp.zeros((_N // 16, _M), _F32)

    enc_ws = [w for w, _ in enc]
    enc_bs = [b.reshape(1, -1) for _, b in enc]
    dec_ws = [w for w, _ in dec[:-1]]
    dec_bs = [b.reshape(1, -1) for _, b in dec[:-1]]
    wlast, blast = dec[-1]
    out_ch = wlast.shape[1]
    dec_ws.append(jnp.pad(wlast, ((0, 0), (0, _C - out_ch))))
    dec_bs.append(jnp.pad(blast, (0, _C - out_ch)).reshape(1, -1))

    msplit = []
    for s in range(steps):
        w1, b1 = msg[s][0]
        msplit.append((w1[:_C], w1[_C:], b1.reshape(1, -1)))

    eye4 = jnp.eye(_PACK, dtype=_F32)
    mtail = []
    for s in range(steps):
        packed = []
        for w, b in msg[s][1:]:
            packed.append(jnp.kron(eye4, w))
            packed.append(jnp.tile(b, _PACK).reshape(1, -1))
        mtail.append(packed)

    h, p, q = _encode(inputs, enc_ws, enc_bs, *msplit[0])
    for s in range(steps):
        xa, xb = _gather2(p, q, src2, dst2)
        xa4 = xa.reshape(_E // _PACK, _M * _PACK)
        xb4 = xb.reshape(_E // _PACK, _M * _PACK)
        m4 = _edge_mlp(xa4, xb4, *mtail[s])
        agg2 = _segment_sum2(m4.reshape(_E, _M), dst3, zrows)
        upd_ws = [w for w, _ in upd[s]]
        upd_bs = [b.reshape(1, -1) for _, b in upd[s]]
        v1a = upd_ws[0][:_C]
        v1b = upd_ws[0][_C:]
        nxt = msplit[s + 1] if s + 1 < steps else None
        res = _update(h, agg2, upd_ws, upd_bs, v1a, v1b, upd_bs[0], nxt)
        if nxt is not None:
            h, p, q = res
        else:
            (h,) = res
    out = _decode(h, dec_ws, dec_bs)
    return out[:, :out_ch]
